# baseline (speedup 1.0000x reference)
import math

import jax
import jax.numpy as jnp
from jax.experimental import pallas as pl
from jax.experimental.pallas import tpu as pltpu

N, T, D = 2048, 20, 512
NEWS = 2048
G, EPG, H, TOPK = 4, 16, 8, 2
HID = G * EPG
HD = EPG // H

BN = 128
NB = N // BN
NBUF = 3
P = 4
CH = BN // P

_NT = (((1,), (1,)), ((), ()))


def _process_block(price, news, mask, w1, b1, w2, b2, wet, be,
                   wq, bq, wk, bk, wv, bv, wo, bo):
    f32 = jnp.float32

    psum = price[:, 0:D]
    nsum = news[:, 0:D] * mask[:, 0:1]
    for t in range(1, T):
        psum = psum + price[:, t * D:(t + 1) * D]
        nsum = nsum + news[:, t * D:(t + 1) * D] * mask[:, t:t + 1]
    p = psum * (1.0 / T)
    msum = jnp.clip(jnp.sum(mask, axis=1, keepdims=True), 1e-6, None)
    n = nsum / msum

    ph = jnp.concatenate([p, n], axis=1)
    h1 = jax.lax.dot_general(ph, w1, _NT, preferred_element_type=f32)
    h1 = jnp.maximum(h1 + b1, 0.0)
    hidden = jax.lax.dot_general(h1, w2, _NT, preferred_element_type=f32) + b2

    lane = jax.lax.broadcasted_iota(jnp.int32, (BN, HID), 1)
    v1 = jnp.max(hidden, axis=1, keepdims=True)
    i1 = jnp.min(jnp.where(hidden == v1, lane, HID), axis=1, keepdims=True)
    rest = jnp.where(lane == i1, -jnp.inf, hidden)
    v2 = jnp.max(rest, axis=1, keepdims=True)
    i2 = jnp.min(jnp.where(rest == v2, lane, HID), axis=1, keepdims=True)
    topmask = (lane == i1) | (lane == i2)
    ew = jnp.where(topmask, jnp.exp(hidden - v1), 0.0)
    rw = ew / jnp.sum(ew, axis=1, keepdims=True)
    tk = jnp.concatenate([i1, i2], axis=1)

    go = jnp.dot(hidden, wet, preferred_element_type=f32) + be
    q = jnp.dot(go, wq, preferred_element_type=f32) + bq
    k = jnp.dot(go, wk, preferred_element_type=f32) + bk
    v = jnp.dot(go, wv, preferred_element_type=f32) + bv

    row = jax.lax.broadcasted_iota(jnp.int32, (HID, HID), 0)
    col = jax.lax.broadcasted_iota(jnp.int32, (HID, HID), 1)
    same_pair = (row // 2) == (col // 2)
    row_even = (row % 2) == 0
    same_grp = (row // EPG) == (col // EPG)
    swap = jnp.where(same_pair & ((row % 2) != (col % 2)), 1.0, 0.0)
    ge = jnp.where(same_grp & row_even, 1.0, 0.0)
    gob = jnp.where(same_grp & ~row_even, 1.0, 0.0)
    pair0 = jnp.where(same_pair & row_even, 1.0, 0.0)
    pair1 = jnp.where(same_pair & ~row_even, 1.0, 0.0)

    pa = q * k
    pb = q * jnp.dot(k, swap, preferred_element_type=f32)
    scale = 1.0 / math.sqrt(HD)
    s00 = jnp.dot(pa, ge, preferred_element_type=f32) * scale
    s11 = jnp.dot(pa, gob, preferred_element_type=f32) * scale
    s01 = jnp.dot(pb, ge, preferred_element_type=f32) * scale
    s10 = jnp.dot(pb, gob, preferred_element_type=f32) * scale

    deven = (lane % 2) == 0
    sa = jnp.where(deven, s00, s10)
    sb = jnp.where(deven, s01, s11)
    m = jnp.maximum(sa, sb)
    ea = jnp.exp(sa - m)
    eb = jnp.exp(sb - m)
    z = ea + eb
    av = (ea / z) * jnp.dot(v, pair0, preferred_element_type=f32) \
        + (eb / z) * jnp.dot(v, pair1, preferred_element_type=f32)
    out = jnp.dot(av, wo, preferred_element_type=f32) + bo

    pred = jnp.sum(out * rw, axis=1, keepdims=True)
    return pred, rw, hidden, tk


def _pipelined_kernel(price_hbm, news_hbm, mask_ref,
                      w1_ref, b1_ref, w2_ref, b2_ref,
                      wet_ref, be_ref, wq_ref, bq_ref, wk_ref, bk_ref,
                      wv_ref, bv_ref, wo_ref, bo_ref,
                      pred_ref, rw_ref, hid_ref, tk_ref,
                      pbuf, nbuf, psem, nsem):

    def copies(i, b):
        cps = []
        for c in range(P):
            r0 = i * BN + c * CH
            cps.append(pltpu.make_async_copy(
                price_hbm.at[pl.ds(r0, CH)], pbuf.at[b, pl.ds(c * CH, CH)],
                psem.at[b]))
            cps.append(pltpu.make_async_copy(
                news_hbm.at[pl.ds(r0, CH)], nbuf.at[b, pl.ds(c * CH, CH)],
                nsem.at[b]))
        return cps

    for i in range(NBUF):
        for cp in copies(i, i % NBUF):
            cp.start()

    w1 = w1_ref[...]
    b1 = b1_ref[...]
    w2 = w2_ref[...]
    b2 = b2_ref[...]
    wet = wet_ref[...]
    be = be_ref[...]
    wq = wq_ref[...]
    bq = bq_ref[...]
    wk = wk_ref[...]
    bk = bk_ref[...]
    wv = wv_ref[...]
    bv = bv_ref[...]
    wo = wo_ref[...]
    bo = bo_ref[...]

    for i in range(NB):
        b = i % NBUF
        for cp in copies(i, b):
            cp.wait()
        mask = mask_ref[pl.ds(i * BN, BN), :]
        pred, rw, hidden, tk = _process_block(
            pbuf[b], nbuf[b], mask, w1, b1, w2, b2, wet, be,
            wq, bq, wk, bk, wv, bv, wo, bo)
        pred_ref[pl.ds(i * BN, BN), :] = pred
        rw_ref[pl.ds(i * BN, BN), :] = rw
        hid_ref[pl.ds(i * BN, BN), :] = hidden
        tk_ref[pl.ds(i * BN, BN), :] = tk
        ni = i + NBUF
        if ni < NB:
            for cp in copies(ni, b):
                cp.start()


def _block_diag_t(w):
    return jax.scipy.linalg.block_diag(*[w[g].T for g in range(G)])


@jax.jit
def kernel(price_feature, news_feature, news_mask, W1, b1, W2, b2, We, be,
           Wq, bq, Wk, bk, Wv, bv, Wo, bo):
    wet = We.reshape(HID, HID).T
    wq_bd = _block_diag_t(Wq)
    wk_bd = _block_diag_t(Wk)
    wv_bd = _block_diag_t(Wv)
    wo_bd = _block_diag_t(Wo)

    vmem = pl.BlockSpec(memory_space=pltpu.MemorySpace.VMEM)
    hbm = pl.BlockSpec(memory_space=pltpu.MemorySpace.HBM)

    out_shapes = (
        jax.ShapeDtypeStruct((N, 1), jnp.float32),
        jax.ShapeDtypeStruct((N, HID), jnp.float32),
        jax.ShapeDtypeStruct((N, HID), jnp.float32),
        jax.ShapeDtypeStruct((N, TOPK), jnp.int32),
    )

    pred, rw, hidden, tk = pl.pallas_call(
        _pipelined_kernel,
        in_specs=[hbm, hbm] + [vmem] * 15,
        out_specs=(vmem, vmem, vmem, vmem),
        out_shape=out_shapes,
        scratch_shapes=[
            pltpu.VMEM((NBUF, BN, T * D), jnp.float32),
            pltpu.VMEM((NBUF, BN, T * D), jnp.float32),
            pltpu.SemaphoreType.DMA((NBUF,)),
            pltpu.SemaphoreType.DMA((NBUF,)),
        ],
    )(price_feature.reshape(N, T * D), news_feature.reshape(N, T * D), news_mask,
      W1, b1.reshape(1, NEWS), W2, b2.reshape(1, HID),
      wet, be.reshape(1, HID), wq_bd, bq.reshape(1, HID),
      wk_bd, bk.reshape(1, HID), wv_bd, bv.reshape(1, HID),
      wo_bd, bo.reshape(1, HID))

    return (pred.reshape(N), rw, hidden, tk, rw)
